# baseline (speedup 1.0000x reference)
import jax
import jax.numpy as jnp
from jax.experimental import pallas as pl
from jax.experimental.pallas import tpu as pltpu

_B = 16384
_T = 1024
_N = _B // _T
_EPS = 1e-7


def _npll_kernel(lh_col_ref, d_col_ref, e_col_ref, lh_row_ref, d_row_ref,
                 out_ref, w_row_ref):
    lh_row = lh_row_ref[:, :]
    gamma = jnp.max(lh_row)
    w_row_ref[:, :] = jnp.exp(lh_row - gamma)

    def outer(ib, num_acc):
        base_i = ib * _T
        d_col = d_col_ref[pl.ds(base_i, _T), :]
        lh_col = lh_col_ref[pl.ds(base_i, _T), :]
        e_col = e_col_ref[pl.ds(base_i, _T), :]
        idx_col = base_i + jax.lax.broadcasted_iota(jnp.int32, (_T, 1), 0)

        def inner(jb, s_acc):
            base_j = jb * _T
            d_r = d_row_ref[:, pl.ds(base_j, _T)]
            w_r = w_row_ref[:, pl.ds(base_j, _T)]
            i_r = base_j + jax.lax.broadcasted_iota(jnp.int32, (1, _T), 1)
            m = (d_r > d_col) | ((d_r == d_col) & (i_r <= idx_col))
            contrib = jnp.sum(jnp.where(m, w_r, 0.0), axis=1, keepdims=True)
            return s_acc + contrib

        s = jax.lax.fori_loop(0, _N, inner, jnp.zeros((_T, 1), jnp.float32))
        term = e_col * (lh_col - (jnp.log(s + _EPS) + gamma))
        return num_acc + jnp.sum(term)

    num = jax.lax.fori_loop(0, _N, outer, jnp.float32(0.0))
    den = jnp.sum(e_col_ref[:, :])
    loss = -num / den
    loss = jnp.where(jnp.isnan(loss), jnp.inf, loss)
    loss = jnp.where(jnp.isneginf(loss), jnp.inf, loss)
    out_ref[0, 0] = loss


def kernel(input, target, weight):
    lh_col = input.reshape(_B, 1)
    d_col = target.reshape(_B, 1)
    e_col = weight.reshape(_B, 1)
    lh_row = input.reshape(1, _B)
    d_row = target.reshape(1, _B)
    out = pl.pallas_call(
        _npll_kernel,
        out_shape=jax.ShapeDtypeStruct((1, 1), jnp.float32),
        out_specs=pl.BlockSpec(memory_space=pltpu.SMEM),
        scratch_shapes=[pltpu.VMEM((1, _B), jnp.float32)],
    )(lh_col, d_col, e_col, lh_row, d_row)
    return out[0, 0]


# speedup vs baseline: 1.3939x; 1.3939x over previous
import functools

import jax
import jax.numpy as jnp
from jax import lax
from jax.experimental import pallas as pl
from jax.experimental.pallas import tpu as pltpu
from jax.experimental.pallas import tpu_sc as plsc

_B = 16384
_L = 16
_CH = _B // _L
_NV = _B // _L
_K = 256
_EPS = 1e-7


def _sc_body(lh_hbm, d_hbm, s_hbm, lhw, dS, key_a, key_b, idx_a, idx_b,
             cnt_a, cnt_b):
    c = lax.axis_index("c")
    s = lax.axis_index("s")

    @pl.when(jnp.logical_and(c == 0, s == 0))
    def _():
        pltpu.sync_copy(lh_hbm, lhw)
        pltpu.sync_copy(d_hbm, dS)

        lane = lax.iota(jnp.int32, _L)
        base = lane * _CH
        ones = jnp.ones((_L,), jnp.int32)
        m255 = jnp.full((_L,), 255, jnp.int32)

        def zero_cnt(cnt):
            def zbody(g, _):
                cnt[pl.ds(g * _L, _L)] = jnp.zeros((_L,), jnp.int32)
                return 0
            lax.fori_loop(0, _K, zbody, 0)

        zero_cnt(cnt_a)
        zero_cnt(cnt_b)

        def gbody(v, m):
            return jnp.maximum(m, lhw[pl.ds(v * _L, _L)])
        mvec = lax.fori_loop(0, _NV, gbody,
                             jnp.full((_L,), -jnp.inf, jnp.float32))
        gamma = plsc.sort_key_val(mvec, mvec)[0][_L - 1]

        def wbody(v, _):
            lhw[pl.ds(v * _L, _L)] = jnp.exp(lhw[pl.ds(v * _L, _L)] - gamma)
            return 0
        lax.fori_loop(0, _NV, wbody, 0)

        def fbody(v, _):
            iv = base + v
            d16 = plsc.load_gather(dS, [iv])
            ub = ~plsc.bitcast(d16, jnp.int32)
            plsc.store_scatter(key_a, [iv], ub)
            plsc.store_scatter(idx_a, [iv], iv)
            dig = ub & m255
            plsc.addupdate_scatter(cnt_a, [dig * _L + lane], ones)
            return 0
        lax.fori_loop(0, _NV, fbody, 0)

        def scan_cnt(cnt):
            def sbody(g, carry):
                row = cnt[pl.ds(g * _L, _L)]
                inc = plsc.cumsum(row)
                cnt[pl.ds(g * _L, _L)] = inc - row + carry
                return carry + inc[_L - 1]
            lax.fori_loop(0, _K, sbody, jnp.int32(0))

        def hist(key_src, cnt, shift):
            sh = jnp.full((_L,), shift, jnp.int32)
            def hbody(v, _):
                k = plsc.load_gather(key_src, [base + v])
                dig = lax.shift_right_logical(k, sh) & m255
                plsc.addupdate_scatter(cnt, [dig * _L + lane], ones)
                return 0
            lax.fori_loop(0, _NV, hbody, 0)

        def permute(key_src, idx_src, key_dst, idx_dst, cnt, shift):
            sh = jnp.full((_L,), shift, jnp.int32)
            def pbody(v, _):
                iv = base + v
                k = plsc.load_gather(key_src, [iv])
                pay = plsc.load_gather(idx_src, [iv])
                dig = lax.shift_right_logical(k, sh) & m255
                slot = dig * _L + lane
                pos = plsc.load_gather(cnt, [slot])
                plsc.store_scatter(key_dst, [pos], k)
                plsc.store_scatter(idx_dst, [pos], pay)
                plsc.addupdate_scatter(cnt, [slot], ones)
                return 0
            lax.fori_loop(0, _NV, pbody, 0)

        scan_cnt(cnt_a)
        permute(key_a, idx_a, key_b, idx_b, cnt_a, 0)

        hist(key_b, cnt_b, 8)
        scan_cnt(cnt_b)
        permute(key_b, idx_b, key_a, idx_a, cnt_b, 8)

        zero_cnt(cnt_a)
        hist(key_a, cnt_a, 16)
        scan_cnt(cnt_a)
        permute(key_a, idx_a, key_b, idx_b, cnt_a, 16)

        zero_cnt(cnt_b)
        hist(key_b, cnt_b, 24)
        scan_cnt(cnt_b)
        permute(key_b, idx_b, key_a, idx_a, cnt_b, 24)

        def cbody(v, carry):
            s16 = idx_a[pl.ds(v * _L, _L)]
            wv = plsc.load_gather(lhw, [s16])
            cs = plsc.cumsum(wv)
            plsc.store_scatter(dS, [s16], cs + carry)
            return carry + cs[_L - 1]
        lax.fori_loop(0, _NV, cbody, jnp.float32(0.0))

        pltpu.sync_copy(dS, s_hbm)


def _risk_set_sums(lh, d):
    mesh = plsc.VectorSubcoreMesh(core_axis_name="c", subcore_axis_name="s")
    return pl.kernel(
        _sc_body,
        out_type=jax.ShapeDtypeStruct((_B,), jnp.float32),
        mesh=mesh,
        compiler_params=pltpu.CompilerParams(needs_layout_passes=False),
        scratch_types=[
            pltpu.VMEM((_B,), jnp.float32),
            pltpu.VMEM((_B,), jnp.float32),
            pltpu.VMEM((_B,), jnp.int32),
            pltpu.VMEM((_B,), jnp.int32),
            pltpu.VMEM((_B,), jnp.int32),
            pltpu.VMEM((_B,), jnp.int32),
            pltpu.VMEM((_K * _L,), jnp.int32),
            pltpu.VMEM((_K * _L,), jnp.int32),
        ],
    )(lh, d)


def _loss_kernel(lh_ref, e_ref, s_ref, out_ref):
    lh = lh_ref[:, :]
    e = e_ref[:, :]
    srow = s_ref[:, :]
    gamma = jnp.max(lh)
    num = jnp.sum(e * (lh - (jnp.log(srow + _EPS) + gamma)))
    den = jnp.sum(e)
    loss = -num / den
    loss = jnp.where(jnp.isnan(loss), jnp.inf, loss)
    loss = jnp.where(jnp.isneginf(loss), jnp.inf, loss)
    out_ref[0, 0] = loss


def kernel(input, target, weight):
    s = _risk_set_sums(input, target)
    out = pl.pallas_call(
        _loss_kernel,
        out_shape=jax.ShapeDtypeStruct((1, 1), jnp.float32),
        out_specs=pl.BlockSpec(memory_space=pltpu.SMEM),
    )(input.reshape(1, _B), weight.reshape(1, _B), s.reshape(1, _B))
    return out[0, 0]
